# Initial kernel scaffold; baseline (speedup 1.0000x reference)
import functools

import jax
import jax.numpy as jnp
from jax import lax
from jax.experimental import pallas as pl
from jax.experimental.pallas import tpu as pltpu
from jax.experimental.pallas import tpu_sc as plsc

R = 4
D = 128
LEAK = 0.01

N = 10000
E = 160000
TBL = 40960

NC = 2
NS = 16
NW = NC * NS
CH = 320
NCHUNK = E // CH
CPW = -(-NCHUNK // NW)

BE = 3200
GRID_E = E // BE
ROWS_PT = N // NS


def _leaky(x):
    return jnp.where(x >= 0, x, LEAK * x)


def _k0_body(xs_ref, xd_ref, ef_ref, t_ref, ti_ref,
             bs0, bs1, bo0, bo1, cs0, cs1, co0, co1,
             atts, atto, attrs, attro, w_ref, aws_ref, abs_ref, awo_ref,
             abo_ref,
             rel_o, msub_o, mobj_o, es_o, eo_o):
    f32 = jnp.float32
    xs = xs_ref[...]
    xd = xd_ref[...]
    tf = t_ref[...]
    tif = ti_ref[...]

    def sel(tab_ref, typ, col):
        out = jnp.zeros_like(typ)
        for r in range(R):
            out = out + jnp.where(typ == float(r), tab_ref[r, col], 0.0)
        return out

    zs0 = jnp.dot(xs, bs0[...], preferred_element_type=f32)
    zs1 = jnp.dot(xs, bs1[...], preferred_element_type=f32)
    zd0 = jnp.dot(xd, bo0[...], preferred_element_type=f32)
    zd1 = jnp.dot(xd, bo1[...], preferred_element_type=f32)
    sub_msg = sel(atts, tf, 0) * zs0 + sel(atts, tf, 1) * zs1
    obj_msg = sel(atto, tif, 0) * zd0 + sel(atto, tif, 1) * zd1

    w = w_ref[...]
    ss = _leaky(jnp.dot(sub_msg, w, preferred_element_type=f32))
    oo = _leaky(jnp.dot(obj_msg, w, preferred_element_type=f32))
    mx = jnp.maximum(ss, oo)
    e0 = jnp.exp(ss - mx)
    e1 = jnp.exp(oo - mx)
    inv = 1.0 / (e0 + e1)
    rel = ef_ref[...] + (e0 * inv) * sub_msg + (e1 * inv) * obj_msg
    rel_o[...] = rel

    def direction(c0, c1, att, aw, ab, typ, m_o, e_out):
        u0 = jnp.dot(rel, c0[...], preferred_element_type=f32)
        u1 = jnp.dot(rel, c1[...], preferred_element_type=f32)
        m = sel(att, typ, 0) * u0 + sel(att, typ, 1) * u1
        m_o[...] = m
        p = jnp.dot(m, aw[...], preferred_element_type=f32)
        score = jnp.zeros_like(typ)
        for r in range(R):
            score = score + jnp.where(typ == float(r),
                                      p[:, r:r + 1] + ab[r, 0], 0.0)
        e_out[...] = jnp.exp(_leaky(score))

    direction(cs0, cs1, attrs, aws_ref, abs_ref, tif, msub_o, es_o)
    direction(co0, co1, attro, awo_ref, abo_ref, tf, mobj_o, eo_o)


def _k0(xs, xd, ef, tf, tif, weights):
    f32 = jnp.float32
    wspecs = [pl.BlockSpec(w.shape, (lambda nd: lambda i: (0,) * nd)(w.ndim))
              for w in weights]
    eb = pl.BlockSpec((BE, D), lambda i: (i, 0))
    sb = pl.BlockSpec((BE, 1), lambda i: (i, 0))
    return pl.pallas_call(
        _k0_body,
        grid=(GRID_E,),
        in_specs=[eb, eb, eb, sb, sb] + wspecs,
        out_specs=[eb, eb, eb, sb, sb],
        out_shape=[jax.ShapeDtypeStruct((E, D), f32)] * 3
        + [jax.ShapeDtypeStruct((E, 1), f32)] * 2,
    )(xs, xd, ef, tf, tif, *weights)


def _g1(nf, src, dst):
    mesh = plsc.VectorSubcoreMesh(core_axis_name="c", subcore_axis_name="s")

    @functools.partial(
        pl.kernel, mesh=mesh,
        out_type=[jax.ShapeDtypeStruct((E, D), jnp.float32)] * 2,
        scratch_types=[
            pltpu.VMEM((CH,), jnp.int32),
            pltpu.VMEM((CH, D), jnp.float32),
            pltpu.SemaphoreType.DMA,
        ],
    )
    def body(nf_hbm, src_hbm, dst_hbm, xs_hbm, xd_hbm, idx_v, rows_v, sem):
        wid = lax.axis_index("s") * NC + lax.axis_index("c")

        def chunk(j, _):
            ck = wid + j * NW

            @pl.when(ck < NCHUNK)
            def _():
                base = ck * CH
                for src_ref, out_ref in ((src_hbm, xs_hbm), (dst_hbm, xd_hbm)):
                    pltpu.sync_copy(src_ref.at[pl.ds(base, CH)], idx_v)
                    pltpu.async_copy(nf_hbm.at[idx_v], rows_v, sem).wait()
                    pltpu.sync_copy(rows_v, out_ref.at[pl.ds(base, CH)])
            return 0

        lax.fori_loop(0, CPW, chunk, 0)

    return body(nf, src, dst)


def _s2(src, dst, t, ti, es, eo):
    mesh = plsc.VectorSubcoreMesh(core_axis_name="c", subcore_axis_name="s")

    @functools.partial(
        pl.kernel, mesh=mesh,
        out_type=[jax.ShapeDtypeStruct((NC, TBL), jnp.float32)] * 2,
        scratch_types=[
            pltpu.VMEM((CH,), jnp.int32),
            pltpu.VMEM((CH,), jnp.int32),
            pltpu.VMEM((CH,), jnp.int32),
            pltpu.VMEM((CH,), jnp.float32),
            pltpu.VMEM((TBL // NS,), jnp.float32),
            pltpu.VMEM_SHARED((TBL,), jnp.float32),
            pltpu.VMEM_SHARED((TBL,), jnp.float32),
        ],
    )
    def body(src_hbm, dst_hbm, t_hbm, ti_hbm, es_hbm, eo_hbm,
             tbs_hbm, tbo_hbm, idx_v, typ_v, key_v, val_v, zb_v, shs, sho):
        c = lax.axis_index("c")
        s = lax.axis_index("s")
        wid = s * NC + c

        def zloop(i, _):
            zb_v[pl.ds(i * 16, 16)] = jnp.zeros((16,), jnp.float32)
            return 0
        lax.fori_loop(0, TBL // NS // 16, zloop, 0)
        pltpu.sync_copy(zb_v, shs.at[pl.ds(s * (TBL // NS), TBL // NS)])
        pltpu.sync_copy(zb_v, sho.at[pl.ds(s * (TBL // NS), TBL // NS)])
        plsc.subcore_barrier()

        def chunk(j, _):
            ck = wid + j * NW

            @pl.when(ck < NCHUNK)
            def _():
                base = ck * CH
                for i_hbm, ty_hbm, v_hbm, sh in (
                        (src_hbm, ti_hbm, es_hbm, shs),
                        (dst_hbm, t_hbm, eo_hbm, sho)):
                    pltpu.sync_copy(i_hbm.at[pl.ds(base, CH)], idx_v)
                    pltpu.sync_copy(ty_hbm.at[pl.ds(base, CH)], typ_v)
                    pltpu.sync_copy(v_hbm.at[pl.ds(base, CH)], val_v)
                    for k in range(CH // 16):
                        sl = pl.ds(k * 16, 16)
                        key_v[sl] = idx_v[sl] * R + typ_v[sl]
                    pltpu.sync_copy(val_v, sh.at[key_v], add=True)
            return 0

        lax.fori_loop(0, CPW, chunk, 0)
        plsc.subcore_barrier()

        sl = pl.ds(s * (TBL // NS), TBL // NS)
        pltpu.sync_copy(shs.at[sl], tbs_hbm.at[c, sl])
        pltpu.sync_copy(sho.at[sl], tbo_hbm.at[c, sl])

    return body(src, dst, t, ti, es, eo)


def _kt_body(tbs_ref, tbo_ref, sums_o, invs_o, sumo_o, invo_o):
    ss = tbs_ref[0:1, :] + tbs_ref[1:2, :]
    so = tbo_ref[0:1, :] + tbo_ref[1:2, :]
    sums_o[...] = ss
    sumo_o[...] = so
    invs_o[...] = 1.0 / (ss + 1e-16)
    invo_o[...] = 1.0 / (so + 1e-16)


def _kt(tbs, tbo):
    f32 = jnp.float32
    return pl.pallas_call(
        _kt_body,
        out_shape=[jax.ShapeDtypeStruct((1, TBL), f32)] * 4,
    )(tbs, tbo)


def _g3(invs, invo, src, dst, t, ti, es, eo):
    mesh = plsc.VectorSubcoreMesh(core_axis_name="c", subcore_axis_name="s")

    @functools.partial(
        pl.kernel, mesh=mesh,
        out_type=[jax.ShapeDtypeStruct((E,), jnp.float32)] * 2,
        scratch_types=[
            pltpu.VMEM((TBL,), jnp.float32),
            pltpu.VMEM((TBL,), jnp.float32),
            pltpu.VMEM((CH,), jnp.int32),
            pltpu.VMEM((CH,), jnp.int32),
            pltpu.VMEM((CH,), jnp.float32),
            pltpu.VMEM((CH,), jnp.float32),
        ],
    )
    def body(invs_hbm, invo_hbm, src_hbm, dst_hbm, t_hbm, ti_hbm,
             es_hbm, eo_hbm, as_hbm, ao_hbm,
             tbs_v, tbo_v, idx_v, typ_v, val_v, out_v):
        wid = lax.axis_index("s") * NC + lax.axis_index("c")
        pltpu.sync_copy(invs_hbm, tbs_v)
        pltpu.sync_copy(invo_hbm, tbo_v)

        def chunk(j, _):
            ck = wid + j * NW

            @pl.when(ck < NCHUNK)
            def _():
                base = ck * CH
                for i_hbm, ty_hbm, v_hbm, tb_v, o_hbm in (
                        (src_hbm, ti_hbm, es_hbm, tbs_v, as_hbm),
                        (dst_hbm, t_hbm, eo_hbm, tbo_v, ao_hbm)):
                    pltpu.sync_copy(i_hbm.at[pl.ds(base, CH)], idx_v)
                    pltpu.sync_copy(ty_hbm.at[pl.ds(base, CH)], typ_v)
                    pltpu.sync_copy(v_hbm.at[pl.ds(base, CH)], val_v)
                    for k in range(CH // 16):
                        sl = pl.ds(k * 16, 16)
                        key = idx_v[sl] * R + typ_v[sl]
                        iv = plsc.load_gather(tb_v, [key])
                        out_v[sl] = val_v[sl] * iv
                    pltpu.sync_copy(out_v, o_hbm.at[pl.ds(base, CH)])
            return 0

        lax.fori_loop(0, CPW, chunk, 0)

    return body(invs, invo, src, dst, t, ti, es, eo)


def _k4_body(msub_ref, asub_ref, mobj_ref, aobj_ref, wsub_o, wobj_o):
    wsub_o[...] = msub_ref[...] * asub_ref[...]
    wobj_o[...] = mobj_ref[...] * aobj_ref[...]


def _k4(msub, asub, mobj, aobj):
    f32 = jnp.float32
    eb = pl.BlockSpec((BE, D), lambda i: (i, 0))
    sb = pl.BlockSpec((BE, 1), lambda i: (i, 0))
    return pl.pallas_call(
        _k4_body,
        grid=(GRID_E,),
        in_specs=[eb, sb, eb, sb],
        out_specs=[eb, eb],
        out_shape=[jax.ShapeDtypeStruct((E, D), f32)] * 2,
    )(msub, asub, mobj, aobj)


def _s5(wsub, wobj, src, dst):
    mesh = plsc.VectorSubcoreMesh(core_axis_name="c", subcore_axis_name="s")

    @functools.partial(
        pl.kernel, mesh=mesh,
        out_type=[jax.ShapeDtypeStruct((N, D), jnp.float32)] * 2,
        scratch_types=[
            pltpu.VMEM((CH,), jnp.int32),
            pltpu.VMEM((CH, D), jnp.float32),
            pltpu.VMEM((ROWS_PT, D), jnp.float32),
            pltpu.VMEM_SHARED((N, D), jnp.float32),
        ],
    )
    def body(wsub_hbm, wobj_hbm, src_hbm, dst_hbm, p0_hbm, p1_hbm,
             idx_v, rows_v, zb_v, acc):
        c = lax.axis_index("c")
        s = lax.axis_index("s")
        wid = s * NC + c

        def zloop(i, _):
            r = i // (D // 16)
            col = (i % (D // 16)) * 16
            zb_v[r, pl.ds(col, 16)] = jnp.zeros((16,), jnp.float32)
            return 0
        lax.fori_loop(0, ROWS_PT * (D // 16), zloop, 0)
        pltpu.sync_copy(zb_v, acc.at[pl.ds(s * ROWS_PT, ROWS_PT)])
        plsc.subcore_barrier()

        def chunk(j, _):
            ck = wid + j * NW

            @pl.when(ck < NCHUNK)
            def _():
                base = ck * CH
                for w_hbm, i_hbm in ((wsub_hbm, src_hbm), (wobj_hbm, dst_hbm)):
                    pltpu.sync_copy(i_hbm.at[pl.ds(base, CH)], idx_v)
                    pltpu.sync_copy(w_hbm.at[pl.ds(base, CH)], rows_v)
                    pltpu.sync_copy(rows_v, acc.at[idx_v], add=True)
            return 0

        lax.fori_loop(0, CPW, chunk, 0)
        plsc.subcore_barrier()

        sl = pl.ds(s * ROWS_PT, ROWS_PT)

        @pl.when(c == 0)
        def _():
            pltpu.sync_copy(acc.at[sl], p0_hbm.at[sl])

        @pl.when(c == 1)
        def _():
            pltpu.sync_copy(acc.at[sl], p1_hbm.at[sl])

    return body(wsub, wobj, src, dst)


def _k6_body(p0_ref, p1_ref, ts_ref, to_ref, out_o):
    tot = ts_ref[...] + to_ref[...]
    active = jnp.sum((tot > 0).astype(jnp.float32), axis=1, keepdims=True)
    out_o[...] = (p0_ref[...] + p1_ref[...]) / jnp.maximum(active, 1.0)


def _k6(p0, p1, sums, sumo):
    f32 = jnp.float32
    nb = pl.BlockSpec((1000, D), lambda i: (i, 0))
    tb = pl.BlockSpec((1000, R), lambda i: (i, 0))
    return pl.pallas_call(
        _k6_body,
        grid=(N // 1000,),
        in_specs=[nb, nb, tb, tb],
        out_specs=nb,
        out_shape=jax.ShapeDtypeStruct((N, D), f32),
    )(p0, p1, sums, sumo)


def kernel(nf, ef, edge_index, edge_type_rel, edge_type_rel_inv,
           sub2rel_basis, sub2rel_att, obj2rel_basis, obj2rel_att,
           rel2sub_basis, rel2sub_att, rel2obj_basis, rel2obj_att,
           entity2rel_w, rel2sub_attn_w, rel2sub_attn_b,
           rel2obj_attn_w, rel2obj_attn_b):
    f32 = jnp.float32
    src = edge_index[0]
    dst = edge_index[1]
    t = edge_type_rel.astype(jnp.int32)
    ti = edge_type_rel_inv.astype(jnp.int32)
    tf = t.astype(f32).reshape(E, 1)
    tif = ti.astype(f32).reshape(E, 1)

    xs, xd = _g1(nf, src, dst)

    weights = [
        sub2rel_basis[0], sub2rel_basis[1],
        obj2rel_basis[0], obj2rel_basis[1],
        rel2sub_basis[0], rel2sub_basis[1],
        rel2obj_basis[0], rel2obj_basis[1],
        sub2rel_att, obj2rel_att, rel2sub_att, rel2obj_att,
        entity2rel_w,
        rel2sub_attn_w.reshape(R, D).T, rel2sub_attn_b,
        rel2obj_attn_w.reshape(R, D).T, rel2obj_attn_b,
    ]
    rel_emb, m_sub, m_obj, es2, eo2 = _k0(xs, xd, ef, tf, tif, weights)
    es = es2.reshape(E)
    eo = eo2.reshape(E)

    tbs, tbo = _s2(src, dst, t, ti, es, eo)
    sums, invs, sumo, invo = _kt(tbs, tbo)
    a_sub, a_obj = _g3(invs.reshape(TBL), invo.reshape(TBL),
                       src, dst, t, ti, es, eo)
    wsub, wobj = _k4(m_sub, a_sub.reshape(E, 1), m_obj, a_obj.reshape(E, 1))
    p0, p1 = _s5(wsub, wobj, src, dst)
    node_out = _k6(p0, p1, sums.reshape(TBL // R, R)[:N],
                   sumo.reshape(TBL // R, R)[:N])
    return node_out, rel_emb


# speedup vs baseline: 16.0494x; 16.0494x over previous
import functools

import jax
import jax.numpy as jnp
from jax import lax
from jax.experimental import pallas as pl
from jax.experimental.pallas import tpu as pltpu
from jax.experimental.pallas import tpu_sc as plsc

R = 4
D = 128
D2 = 64
LEAK = 0.01

N = 10000
E = 160000
TBL = 40960

NC = 2
NS = 16
NW = NC * NS
CH = 320
NCHUNK = E // CH
CPW = -(-NCHUNK // NW)
CPW1 = -(-NCHUNK // NS)

BE = 3200
GRID_E = E // BE
NHALF = N // 2
ACC_ROWS = 5248
ZROWS = ACC_ROWS // NS
DUMP_ROW = 5200


def _leaky(x):
    return jnp.where(x >= 0, x, LEAK * x)


def _k0_body(xs_ref, xd_ref, ef_ref, t_ref, ti_ref,
             bs0, bs1, bo0, bo1, cs0, cs1, co0, co1,
             atts, atto, attrs, attro, w_ref, aws_ref, abs_ref, awo_ref,
             abo_ref,
             rel_o, msub_o, mobj_o, es_o, eo_o):
    f32 = jnp.float32
    xs = xs_ref[...]
    xd = xd_ref[...]
    tf = t_ref[...]
    tif = ti_ref[...]

    def sel(tab_ref, typ, col):
        out = jnp.zeros_like(typ)
        for r in range(R):
            out = out + jnp.where(typ == float(r), tab_ref[r, col], 0.0)
        return out

    zs0 = jnp.dot(xs, bs0[...], preferred_element_type=f32)
    zs1 = jnp.dot(xs, bs1[...], preferred_element_type=f32)
    zd0 = jnp.dot(xd, bo0[...], preferred_element_type=f32)
    zd1 = jnp.dot(xd, bo1[...], preferred_element_type=f32)
    sub_msg = sel(atts, tf, 0) * zs0 + sel(atts, tf, 1) * zs1
    obj_msg = sel(atto, tif, 0) * zd0 + sel(atto, tif, 1) * zd1

    w = w_ref[...]
    ss = _leaky(jnp.dot(sub_msg, w, preferred_element_type=f32))
    oo = _leaky(jnp.dot(obj_msg, w, preferred_element_type=f32))
    mx = jnp.maximum(ss, oo)
    e0 = jnp.exp(ss - mx)
    e1 = jnp.exp(oo - mx)
    inv = 1.0 / (e0 + e1)
    rel = ef_ref[...] + (e0 * inv) * sub_msg + (e1 * inv) * obj_msg
    rel_o[...] = rel

    def direction(c0, c1, att, aw, ab, typ, m_o, e_out):
        u0 = jnp.dot(rel, c0[...], preferred_element_type=f32)
        u1 = jnp.dot(rel, c1[...], preferred_element_type=f32)
        m = sel(att, typ, 0) * u0 + sel(att, typ, 1) * u1
        m_o[...] = m
        p = jnp.dot(m, aw[...], preferred_element_type=f32)
        score = jnp.zeros_like(typ)
        for r in range(R):
            score = score + jnp.where(typ == float(r),
                                      p[:, r:r + 1] + ab[r, 0], 0.0)
        e_out[...] = jnp.exp(_leaky(score))

    direction(cs0, cs1, attrs, aws_ref, abs_ref, tif, msub_o, es_o)
    direction(co0, co1, attro, awo_ref, abo_ref, tf, mobj_o, eo_o)


def _k0(xs, xd, ef, tf, tif, weights):
    f32 = jnp.float32
    wspecs = [pl.BlockSpec(w.shape, (lambda nd: lambda i: (0,) * nd)(w.ndim))
              for w in weights]
    eb = pl.BlockSpec((BE, D), lambda i: (i, 0))
    sb = pl.BlockSpec((BE, 1), lambda i: (i, 0))
    return pl.pallas_call(
        _k0_body,
        grid=(GRID_E,),
        in_specs=[eb, eb, eb, sb, sb] + wspecs,
        out_specs=[eb, eb, eb, sb, sb],
        out_shape=[jax.ShapeDtypeStruct((E, D), f32)] * 3
        + [jax.ShapeDtypeStruct((E, 1), f32)] * 2,
    )(xs, xd, ef, tf, tif, *weights)


def _g1(nf, src, dst):
    mesh = plsc.VectorSubcoreMesh(core_axis_name="c", subcore_axis_name="s")

    @functools.partial(
        pl.kernel, mesh=mesh,
        out_type=[jax.ShapeDtypeStruct((E, D), jnp.float32)] * 2,
        scratch_types=[
            pltpu.VMEM((CH,), jnp.int32),
            pltpu.VMEM((CH, D), jnp.float32),
            pltpu.SemaphoreType.DMA,
        ],
    )
    def body(nf_hbm, src_hbm, dst_hbm, xs_hbm, xd_hbm, idx_v, rows_v, sem):
        wid = lax.axis_index("s") * NC + lax.axis_index("c")

        def chunk(j, _):
            ck = wid + j * NW

            @pl.when(ck < NCHUNK)
            def _():
                base = ck * CH
                for src_ref, out_ref in ((src_hbm, xs_hbm), (dst_hbm, xd_hbm)):
                    pltpu.sync_copy(src_ref.at[pl.ds(base, CH)], idx_v)
                    pltpu.async_copy(nf_hbm.at[idx_v], rows_v, sem).wait()
                    pltpu.sync_copy(rows_v, out_ref.at[pl.ds(base, CH)])
            return 0

        lax.fori_loop(0, CPW, chunk, 0)

    return body(nf, src, dst)


def _s2(src, dst, t, ti, es, eo):
    mesh = plsc.VectorSubcoreMesh(core_axis_name="c", subcore_axis_name="s")

    @functools.partial(
        pl.kernel, mesh=mesh,
        out_type=[jax.ShapeDtypeStruct((NW * TBL,), jnp.float32)] * 2,
        compiler_params=pltpu.CompilerParams(needs_layout_passes=False),
        scratch_types=[
            pltpu.VMEM((CH,), jnp.int32),
            pltpu.VMEM((CH,), jnp.int32),
            pltpu.VMEM((CH,), jnp.float32),
            pltpu.VMEM((TBL,), jnp.float32),
            pltpu.VMEM((TBL,), jnp.float32),
        ],
    )
    def body(src_hbm, dst_hbm, t_hbm, ti_hbm, es_hbm, eo_hbm,
             tbs_hbm, tbo_hbm, idx_v, typ_v, val_v, tbs_v, tbo_v):
        c = lax.axis_index("c")
        s_ = lax.axis_index("s")
        wid = s_ * NC + c

        def zloop(i, _):
            tbs_v[pl.ds(i * 16, 16)] = jnp.zeros((16,), jnp.float32)
            tbo_v[pl.ds(i * 16, 16)] = jnp.zeros((16,), jnp.float32)
            return 0
        lax.fori_loop(0, TBL // 16, zloop, 0)

        def chunk(j, _):
            ck = wid + j * NW

            @pl.when(ck < NCHUNK)
            def _():
                base = ck * CH
                for i_hbm, ty_hbm, v_hbm, tb_v in (
                        (src_hbm, ti_hbm, es_hbm, tbs_v),
                        (dst_hbm, t_hbm, eo_hbm, tbo_v)):
                    pltpu.sync_copy(i_hbm.at[pl.ds(base, CH)], idx_v)
                    pltpu.sync_copy(ty_hbm.at[pl.ds(base, CH)], typ_v)
                    pltpu.sync_copy(v_hbm.at[pl.ds(base, CH)], val_v)
                    for k in range(CH // 16):
                        sl = pl.ds(k * 16, 16)
                        key = idx_v[sl] * R + typ_v[sl]
                        plsc.addupdate_scatter(tb_v, [key], val_v[sl])
            return 0

        lax.fori_loop(0, CPW, chunk, 0)

        pltpu.sync_copy(tbs_v, tbs_hbm.at[pl.ds(wid * TBL, TBL)])
        pltpu.sync_copy(tbo_v, tbo_hbm.at[pl.ds(wid * TBL, TBL)])

    return body(src, dst, t, ti, es, eo)


def _kt_body(tbs_ref, tbo_ref, sums_o, invs_o, sumo_o, invo_o):
    ss = jnp.sum(tbs_ref[...], axis=0, keepdims=True)
    so = jnp.sum(tbo_ref[...], axis=0, keepdims=True)
    sums_o[...] = ss
    sumo_o[...] = so
    invs_o[...] = 1.0 / (ss + 1e-16)
    invo_o[...] = 1.0 / (so + 1e-16)


def _kt(tbs, tbo):
    f32 = jnp.float32
    cb = pl.BlockSpec((NW, 4096), lambda i: (0, i))
    ob = pl.BlockSpec((1, 4096), lambda i: (0, i))
    return pl.pallas_call(
        _kt_body,
        grid=(TBL // 4096,),
        in_specs=[cb, cb],
        out_specs=[ob, ob, ob, ob],
        out_shape=[jax.ShapeDtypeStruct((1, TBL), f32)] * 4,
    )(tbs, tbo)


def _g3(invs, invo, src, dst, t, ti, es, eo):
    mesh = plsc.VectorSubcoreMesh(core_axis_name="c", subcore_axis_name="s")

    @functools.partial(
        pl.kernel, mesh=mesh,
        out_type=[jax.ShapeDtypeStruct((E,), jnp.float32)] * 2,
        compiler_params=pltpu.CompilerParams(needs_layout_passes=False),
        scratch_types=[
            pltpu.VMEM((TBL,), jnp.float32),
            pltpu.VMEM((TBL,), jnp.float32),
            pltpu.VMEM((CH,), jnp.int32),
            pltpu.VMEM((CH,), jnp.int32),
            pltpu.VMEM((CH,), jnp.float32),
            pltpu.VMEM((CH,), jnp.float32),
        ],
    )
    def body(invs_hbm, invo_hbm, src_hbm, dst_hbm, t_hbm, ti_hbm,
             es_hbm, eo_hbm, as_hbm, ao_hbm,
             tbs_v, tbo_v, idx_v, typ_v, val_v, out_v):
        wid = lax.axis_index("s") * NC + lax.axis_index("c")
        pltpu.sync_copy(invs_hbm, tbs_v)
        pltpu.sync_copy(invo_hbm, tbo_v)

        def chunk(j, _):
            ck = wid + j * NW

            @pl.when(ck < NCHUNK)
            def _():
                base = ck * CH
                for i_hbm, ty_hbm, v_hbm, tb_v, o_hbm in (
                        (src_hbm, ti_hbm, es_hbm, tbs_v, as_hbm),
                        (dst_hbm, t_hbm, eo_hbm, tbo_v, ao_hbm)):
                    pltpu.sync_copy(i_hbm.at[pl.ds(base, CH)], idx_v)
                    pltpu.sync_copy(ty_hbm.at[pl.ds(base, CH)], typ_v)
                    pltpu.sync_copy(v_hbm.at[pl.ds(base, CH)], val_v)
                    for k in range(CH // 16):
                        sl = pl.ds(k * 16, 16)
                        key = idx_v[sl] * R + typ_v[sl]
                        iv = plsc.load_gather(tb_v, [key])
                        out_v[sl] = val_v[sl] * iv
                    pltpu.sync_copy(out_v, o_hbm.at[pl.ds(base, CH)])
            return 0

        lax.fori_loop(0, CPW, chunk, 0)

    return body(invs, invo, src, dst, t, ti, es, eo)


def _k4_body(msub_ref, asub_ref, mobj_ref, aobj_ref, wsub_o, wobj_o):
    wsub_o[...] = msub_ref[...] * asub_ref[...]
    wobj_o[...] = mobj_ref[...] * aobj_ref[...]


def _k4(msub, asub, mobj, aobj):
    f32 = jnp.float32
    eb = pl.BlockSpec((BE, D), lambda i: (i, 0))
    sb = pl.BlockSpec((BE, 1), lambda i: (i, 0))
    return pl.pallas_call(
        _k4_body,
        grid=(GRID_E,),
        in_specs=[eb, sb, eb, sb],
        out_specs=[eb, eb],
        out_shape=[jax.ShapeDtypeStruct((E, D), f32)] * 2,
    )(msub, asub, mobj, aobj)


def _s5(wsub, wobj, src, dst):
    mesh = plsc.VectorSubcoreMesh(core_axis_name="c", subcore_axis_name="s")

    @functools.partial(
        pl.kernel, mesh=mesh,
        out_type=jax.ShapeDtypeStruct((N, D), jnp.float32),
        scratch_types=[
            pltpu.VMEM((CH,), jnp.int32),
            pltpu.VMEM((CH,), jnp.int32),
            pltpu.VMEM((CH, D), jnp.float32),
            pltpu.VMEM((ZROWS, D), jnp.float32),
            pltpu.VMEM_SHARED((ACC_ROWS, D), jnp.float32),
        ],
    )
    def body(wsub_hbm, wobj_hbm, src_hbm, dst_hbm, psum_hbm,
             idx_v, idx2_v, rows_v, zb_v, acc):
        c = lax.axis_index("c")
        s_ = lax.axis_index("s")
        base_node = c * NHALF

        def zloop(i, _):
            r = i // (D // 16)
            col = (i % (D // 16)) * 16
            zb_v[r, pl.ds(col, 16)] = jnp.zeros((16,), jnp.float32)
            return 0
        lax.fori_loop(0, ZROWS * (D // 16), zloop, 0)
        pltpu.sync_copy(zb_v, acc.at[pl.ds(s_ * ZROWS, ZROWS)])
        plsc.subcore_barrier()

        def chunk(j, _):
            ck = s_ + j * NS

            @pl.when(ck < NCHUNK)
            def _():
                base = ck * CH
                for w_hbm, i_hbm in ((wsub_hbm, src_hbm), (wobj_hbm, dst_hbm)):
                    pltpu.sync_copy(i_hbm.at[pl.ds(base, CH)], idx_v)
                    pltpu.sync_copy(w_hbm.at[pl.ds(base, CH)], rows_v)
                    for k in range(CH // 16):
                        sl = pl.ds(k * 16, 16)
                        lv = idx_v[sl] - base_node
                        keep = (lv >= 0) & (lv < NHALF)
                        idx2_v[sl] = jnp.where(keep, lv, DUMP_ROW)
                    pltpu.sync_copy(rows_v, acc.at[idx2_v], add=True)
            return 0

        lax.fori_loop(0, CPW1, chunk, 0)
        plsc.subcore_barrier()

        row0 = s_ * CH

        @pl.when(row0 + CH <= NHALF)
        def _():
            pltpu.sync_copy(acc.at[pl.ds(row0, CH)], rows_v)
            pltpu.sync_copy(rows_v, psum_hbm.at[pl.ds(base_node + row0, CH)])

        @pl.when(row0 + CH > NHALF)
        def _():
            tail = NHALF % CH
            pltpu.sync_copy(acc.at[pl.ds(row0, tail)], rows_v.at[pl.ds(0, tail)])
            pltpu.sync_copy(rows_v.at[pl.ds(0, tail)],
                            psum_hbm.at[pl.ds(base_node + row0, tail)])

    return body(wsub, wobj, src, dst)


def _k6_body(ps_ref, ts_ref, to_ref, out_o):
    tot = ts_ref[...] + to_ref[...]
    active = jnp.sum((tot > 0).astype(jnp.float32), axis=1, keepdims=True)
    out_o[...] = ps_ref[...] / jnp.maximum(active, 1.0)


def _k6(psum, sums, sumo):
    f32 = jnp.float32
    nb = pl.BlockSpec((1000, D), lambda i: (i, 0))
    tb = pl.BlockSpec((1000, R), lambda i: (i, 0))
    return pl.pallas_call(
        _k6_body,
        grid=(N // 1000,),
        in_specs=[nb, tb, tb],
        out_specs=nb,
        out_shape=jax.ShapeDtypeStruct((N, D), f32),
    )(psum, sums, sumo)


def kernel(nf, ef, edge_index, edge_type_rel, edge_type_rel_inv,
           sub2rel_basis, sub2rel_att, obj2rel_basis, obj2rel_att,
           rel2sub_basis, rel2sub_att, rel2obj_basis, rel2obj_att,
           entity2rel_w, rel2sub_attn_w, rel2sub_attn_b,
           rel2obj_attn_w, rel2obj_attn_b):
    f32 = jnp.float32
    src = edge_index[0]
    dst = edge_index[1]
    t = edge_type_rel.astype(jnp.int32)
    ti = edge_type_rel_inv.astype(jnp.int32)
    tf = t.astype(f32).reshape(E, 1)
    tif = ti.astype(f32).reshape(E, 1)

    xs, xd = _g1(nf, src, dst)

    weights = [
        sub2rel_basis[0], sub2rel_basis[1],
        obj2rel_basis[0], obj2rel_basis[1],
        rel2sub_basis[0], rel2sub_basis[1],
        rel2obj_basis[0], rel2obj_basis[1],
        sub2rel_att, obj2rel_att, rel2sub_att, rel2obj_att,
        entity2rel_w,
        rel2sub_attn_w.reshape(R, D).T, rel2sub_attn_b,
        rel2obj_attn_w.reshape(R, D).T, rel2obj_attn_b,
    ]
    rel_emb, m_sub, m_obj, es2, eo2 = _k0(xs, xd, ef, tf, tif, weights)
    es = es2.reshape(E)
    eo = eo2.reshape(E)

    tbs, tbo = _s2(src, dst, t, ti, es, eo)
    sums, invs, sumo, invo = _kt(tbs.reshape(NW, TBL), tbo.reshape(NW, TBL))
    a_sub, a_obj = _g3(invs.reshape(TBL), invo.reshape(TBL),
                       src, dst, t, ti, es, eo)
    wsub, wobj = _k4(m_sub, a_sub.reshape(E, 1), m_obj, a_obj.reshape(E, 1))
    psum = _s5(wsub, wobj, src, dst)
    node_out = _k6(psum, sums.reshape(TBL // R, R)[:N],
                   sumo.reshape(TBL // R, R)[:N])
    return node_out, rel_emb
